# Initial kernel scaffold; baseline (speedup 1.0000x reference)
import jax
import jax.numpy as jnp
from jax.experimental import pallas as pl


def kernel(h_author, h_paper, edge_index_writes, edge_index_writtenby, Wk, bk, Wq, bq, Wv, bv, Wa, ba, rel_att, rel_msg, rel_pri, skip):
    raise NotImplementedError("write your pallas kernel here")


# speedup vs baseline: 12.4886x; 12.4886x over previous
import functools

import jax
import jax.numpy as jnp
from jax import lax
from jax.experimental import pallas as pl
from jax.experimental.pallas import tpu as pltpu
from jax.experimental.pallas import tpu_sc as plsc

N = 10000
E = 320000
D = 128
HEADS = 8
DK = 16
SQRT_DK = float(DK) ** 0.5
SROWS = 640
SSTRIPE = SROWS // 16

NC = 2
NS = 16
NW = NC * NS
EPW = E // NW
BLK = 80
NBLK = EPW // BLK
STRIPE = 624
TAIL0 = NS * STRIPE
TAIL = N - TAIL0

RB = 1000

_f32 = jnp.float32


def _proj_body(h_ref, w_ref, b_ref, o_ref):
    o_ref[0] = (
        jnp.dot(h_ref[0], w_ref[0], preferred_element_type=_f32) + b_ref[0]
    )


_PW = 3 * D

_proj_call = pl.pallas_call(
    _proj_body,
    grid=(2, N // RB),
    in_specs=[
        pl.BlockSpec((1, RB, D), lambda i, j: (i, j, 0)),
        pl.BlockSpec((1, D, _PW), lambda i, j: (i, 0, 0)),
        pl.BlockSpec((1, 1, _PW), lambda i, j: (i, 0, 0)),
    ],
    out_specs=pl.BlockSpec((1, RB, _PW), lambda i, j: (i, j, 0)),
    out_shape=jax.ShapeDtypeStruct((2, N, _PW), _f32),
)


def _edge_body(k_hbm, q_hbm, v_hbm, src_hbm, dst_hbm, zagg_hbm,
               agg_out, s_out,
               srcv, dstv, dstrow, krows, qrows, wbuf,
               agg_sp, s_sp, sem):
    c = lax.axis_index("c")
    s = lax.axis_index("s")
    wid = s * NC + c

    r0 = s * STRIPE
    rs0 = s * SSTRIPE
    pltpu.sync_copy(zagg_hbm.at[pl.ds(r0, STRIPE)],
                    agg_sp.at[pl.ds(r0, STRIPE)])
    pltpu.sync_copy(zagg_hbm.at[pl.ds(TAIL0, TAIL)],
                    agg_sp.at[pl.ds(TAIL0, TAIL)])
    pltpu.sync_copy(zagg_hbm.at[pl.ds(rs0, SSTRIPE)],
                    s_sp.at[pl.ds(rs0, SSTRIPE)])
    pltpu.sync_copy(zagg_hbm.at[pl.ds(0, BLK)], wbuf)

    plsc.subcore_barrier()

    ebase = wid * EPW

    def blk(b, carry):
        off = ebase + b * BLK
        pltpu.sync_copy(src_hbm.at[pl.ds(off, BLK)], srcv)
        pltpu.sync_copy(dst_hbm.at[pl.ds(off, BLK)], dstv)
        ck = pltpu.async_copy(k_hbm.at[srcv], krows, sem)
        cq = pltpu.async_copy(q_hbm.at[dstv], qrows, sem)
        ck.wait()
        cq.wait()
        for g in range(BLK // DK):
            e16 = jax.lax.iota(jnp.int32, 16) + (g * 16)
            d16 = dstv[pl.ds(g * 16, 16)]
            dstrow[pl.ds(g * 16, 16)] = lax.shift_right_logical(d16, 4)
            cbase16 = (d16 & 15) * HEADS
            for h in range(HEADS):
                def dot_step(dk, acc):
                    col = jnp.full((16,), h * DK, jnp.int32) + dk
                    kg = plsc.load_gather(krows, [e16, col])
                    qg = plsc.load_gather(qrows, [e16, col])
                    return acc + kg * qg
                t = lax.fori_loop(0, DK, dot_step, jnp.zeros((16,), _f32))
                w = jnp.exp(t)
                plsc.store_scatter(wbuf, [e16, cbase16 + h], w)
        pltpu.async_copy(v_hbm.at[srcv], krows, sem).wait()
        for g in range(BLK // DK):
            e16 = jax.lax.iota(jnp.int32, 16) + (g * 16)
            cbase16 = (dstv[pl.ds(g * 16, 16)] & 15) * HEADS
            for h in range(HEADS):
                w = plsc.load_gather(wbuf, [e16, cbase16 + h])
                for dk in range(DK):
                    col = jnp.full((16,), h * DK + dk, jnp.int32)
                    vg = plsc.load_gather(krows, [e16, col])
                    plsc.store_scatter(krows, [e16, col], w * vg)
        pltpu.sync_copy(krows, agg_sp.at[dstv], add=True)
        pltpu.sync_copy(wbuf, s_sp.at[dstrow], add=True)
        zero16 = jnp.zeros((16,), _f32)
        for g in range(BLK // DK):
            e16 = jax.lax.iota(jnp.int32, 16) + (g * 16)
            cbase16 = (dstv[pl.ds(g * 16, 16)] & 15) * HEADS
            for h in range(HEADS):
                plsc.store_scatter(wbuf, [e16, cbase16 + h], zero16)
        return carry

    lax.fori_loop(0, NBLK, blk, 0)
    plsc.subcore_barrier()

    cb = c * N
    cs = c * SROWS
    pltpu.sync_copy(agg_sp.at[pl.ds(r0, STRIPE)],
                    agg_out.at[pl.ds(cb + r0, STRIPE)])
    pltpu.sync_copy(agg_sp.at[pl.ds(TAIL0, TAIL)],
                    agg_out.at[pl.ds(cb + TAIL0, TAIL)])
    pltpu.sync_copy(s_sp.at[pl.ds(rs0, SSTRIPE)],
                    s_out.at[pl.ds(cs + rs0, SSTRIPE)])


@functools.lru_cache(maxsize=1)
def _get_edge_call():
  return pl.kernel(
    _edge_body,
    out_type=(
        jax.ShapeDtypeStruct((NC * N, D), _f32),
        jax.ShapeDtypeStruct((NC * SROWS, D), _f32),
    ),
    mesh=plsc.VectorSubcoreMesh(
        core_axis_name="c", subcore_axis_name="s",
        num_cores=NC, num_subcores=NS),
    scratch_types=[
        pltpu.MemorySpace.VMEM((BLK,), jnp.int32),
        pltpu.MemorySpace.VMEM((BLK,), jnp.int32),
        pltpu.MemorySpace.VMEM((BLK,), jnp.int32),
        pltpu.MemorySpace.VMEM((BLK, D), _f32),
        pltpu.MemorySpace.VMEM((BLK, D), _f32),
        pltpu.MemorySpace.VMEM((BLK, D), _f32),
        pltpu.MemorySpace.VMEM_SHARED((N, D), _f32),
        pltpu.MemorySpace.VMEM_SHARED((SROWS, D), _f32),
        pltpu.SemaphoreType.DMA,
    ],
    compiler_params=pltpu.CompilerParams(needs_layout_passes=False),
  )


def _out_body(a_ref, s_ref, h_ref, w_ref, e_ref, ba_ref, beta_ref, o_ref):
    sfull = jnp.dot(s_ref[0] + s_ref[1], e_ref[...],
                    preferred_element_type=_f32)
    sfull = jnp.where(sfull > 0.0, sfull, 1.0)
    aggn = (a_ref[0] + a_ref[1]) / sfull
    o_ref[...] = (
        jnp.dot(aggn, w_ref[...], preferred_element_type=_f32)
        + ba_ref[...] + h_ref[...] * beta_ref[...]
    )


_out_call = pl.pallas_call(
    _out_body,
    grid=(N // RB,),
    in_specs=[
        pl.BlockSpec((NC, RB, D), lambda i: (0, i, 0)),
        pl.BlockSpec((NC, RB, HEADS), lambda i: (0, i, 0)),
        pl.BlockSpec((RB, D), lambda i: (i, 0)),
        pl.BlockSpec((D, D), lambda i: (0, 0)),
        pl.BlockSpec((HEADS, D), lambda i: (0, 0)),
        pl.BlockSpec((1, D), lambda i: (0, 0)),
        pl.BlockSpec((1, D), lambda i: (0, 0)),
    ],
    out_specs=pl.BlockSpec((RB, D), lambda i: (i, 0)),
    out_shape=jax.ShapeDtypeStruct((N, D), _f32),
)


def _fold(W, b, R):
    We = jnp.einsum('ihj,hjk->ihk', W.reshape(D, HEADS, DK), R).reshape(D, D)
    be = jnp.einsum('hj,hjk->hk', b.reshape(HEADS, DK), R).reshape(D)
    return We, be


def kernel(h_author, h_paper, edge_index_writes, edge_index_writtenby,
           Wk, bk, Wq, bq, Wv, bv, Wa, ba, rel_att, rel_msg, rel_pri, skip):
    h_author = h_author.astype(_f32)
    h_paper = h_paper.astype(_f32)

    Wk0, bk0 = _fold(Wk[0], bk[0], rel_att[0])
    Wv0, bv0 = _fold(Wv[0], bv[0], rel_msg[0])
    Wk1, bk1 = _fold(Wk[1], bk[1], rel_att[1])
    Wv1, bv1 = _fold(Wv[1], bv[1], rel_msg[1])
    scale0 = jnp.repeat(rel_pri[0] / SQRT_DK, DK)
    scale1 = jnp.repeat(rel_pri[1] / SQRT_DK, DK)
    Wq_a = Wq[0] * scale1[None, :]
    bq_a = bq[0] * scale1
    Wq_p = Wq[1] * scale0[None, :]
    bq_p = bq[1] * scale0

    Wstack = jnp.stack([
        jnp.concatenate([Wk0, Wv0, Wq_a], axis=1),
        jnp.concatenate([Wk1, Wv1, Wq_p], axis=1),
    ])
    bstack = jnp.stack([
        jnp.concatenate([bk0, bv0, bq_a]),
        jnp.concatenate([bk1, bv1, bq_p]),
    ])[:, None, :]
    hstack = jnp.stack([h_author, h_paper])

    kvq = _proj_call(hstack, Wstack, bstack)
    K_a, V_a, Q_a = kvq[0, :, :D], kvq[0, :, D:2 * D], kvq[0, :, 2 * D:]
    K_p, V_p, Q_p = kvq[1, :, :D], kvq[1, :, D:2 * D], kvq[1, :, 2 * D:]

    src0 = edge_index_writes[0].astype(jnp.int32)
    dst0 = edge_index_writes[1].astype(jnp.int32)
    src1 = edge_index_writtenby[0].astype(jnp.int32)
    dst1 = edge_index_writtenby[1].astype(jnp.int32)

    zagg = jnp.zeros((N, D), _f32)

    edge_call = _get_edge_call()
    agg_p2, sp_p2 = edge_call(K_a, Q_p, V_a, src0, dst0, zagg)
    agg_a2, sp_a2 = edge_call(K_p, Q_a, V_p, src1, dst1, zagg)
    agg_p2 = agg_p2.reshape(NC, N, D)
    agg_a2 = agg_a2.reshape(NC, N, D)
    s_p2 = sp_p2.reshape(NC, SROWS * 16, HEADS)[:, :N, :]
    s_a2 = sp_a2.reshape(NC, SROWS * 16, HEADS)[:, :N, :]

    emat = jnp.kron(jnp.eye(HEADS, dtype=_f32), jnp.ones((1, DK), _f32))
    alpha_a = jax.nn.sigmoid(skip[0])
    alpha_p = jax.nn.sigmoid(skip[1])
    new_author = _out_call(
        agg_a2, s_a2, h_author, Wa[0] * alpha_a,
        emat, (ba[0] * alpha_a)[None, :],
        jnp.full((1, D), 1.0 - alpha_a, _f32))
    new_paper = _out_call(
        agg_p2, s_p2, h_paper, Wa[1] * alpha_p,
        emat, (ba[1] * alpha_p)[None, :],
        jnp.full((1, D), 1.0 - alpha_p, _f32))
    return (new_author, new_paper)


# speedup vs baseline: 12.9370x; 1.0359x over previous
import functools

import jax
import jax.numpy as jnp
from jax import lax
from jax.experimental import pallas as pl
from jax.experimental.pallas import tpu as pltpu
from jax.experimental.pallas import tpu_sc as plsc

N = 10000
E = 320000
D = 128
HEADS = 8
DK = 16
SQRT_DK = float(DK) ** 0.5
SROWS = 640
SSTRIPE = SROWS // 16

NC = 2
NS = 16
NW = NC * NS
EPW = E // NW
BLK = 80
NBLK = EPW // BLK
STRIPE = 624
TAIL0 = NS * STRIPE
TAIL = N - TAIL0

RB = 1000

_f32 = jnp.float32


def _proj_body(h_ref, w_ref, b_ref, o_ref):
    o_ref[0] = (
        jnp.dot(h_ref[0], w_ref[0], preferred_element_type=_f32) + b_ref[0]
    )


_PW = 3 * D

_proj_call = pl.pallas_call(
    _proj_body,
    grid=(2, N // RB),
    in_specs=[
        pl.BlockSpec((1, RB, D), lambda i, j: (i, j, 0)),
        pl.BlockSpec((1, D, _PW), lambda i, j: (i, 0, 0)),
        pl.BlockSpec((1, 1, _PW), lambda i, j: (i, 0, 0)),
    ],
    out_specs=pl.BlockSpec((1, RB, _PW), lambda i, j: (i, j, 0)),
    out_shape=jax.ShapeDtypeStruct((2, N, _PW), _f32),
)


def _edge_body(k_hbm, q_hbm, v_hbm, src_hbm, dst_hbm, zagg_hbm,
               agg_out, s_out,
               srcv, dstv, dstrow, krows, qrows, vbuf, wbuf,
               agg_sp, s_sp, sem):
    c = lax.axis_index("c")
    s = lax.axis_index("s")
    wid = s * NC + c

    r0 = s * STRIPE
    rs0 = s * SSTRIPE
    pltpu.sync_copy(zagg_hbm.at[pl.ds(r0, STRIPE)],
                    agg_sp.at[pl.ds(r0, STRIPE)])
    pltpu.sync_copy(zagg_hbm.at[pl.ds(TAIL0, TAIL)],
                    agg_sp.at[pl.ds(TAIL0, TAIL)])
    pltpu.sync_copy(zagg_hbm.at[pl.ds(rs0, SSTRIPE)],
                    s_sp.at[pl.ds(rs0, SSTRIPE)])
    pltpu.sync_copy(zagg_hbm.at[pl.ds(0, BLK)], wbuf)

    plsc.subcore_barrier()

    ebase = wid * EPW

    def blk(b, carry):
        off = ebase + b * BLK
        pltpu.sync_copy(src_hbm.at[pl.ds(off, BLK)], srcv)
        pltpu.sync_copy(dst_hbm.at[pl.ds(off, BLK)], dstv)
        ck = pltpu.async_copy(k_hbm.at[srcv], krows, sem)
        cq = pltpu.async_copy(q_hbm.at[dstv], qrows, sem)
        cv = pltpu.async_copy(v_hbm.at[srcv], vbuf, sem)
        ck.wait()
        cq.wait()
        for g in range(BLK // DK):
            e16 = jax.lax.iota(jnp.int32, 16) + (g * 16)
            d16 = dstv[pl.ds(g * 16, 16)]
            dstrow[pl.ds(g * 16, 16)] = lax.shift_right_logical(d16, 4)
            cbase16 = (d16 & 15) * HEADS
            for h in range(HEADS):
                def dot_step(dk, acc):
                    col = jnp.full((16,), h * DK, jnp.int32) + dk
                    kg = plsc.load_gather(krows, [e16, col])
                    qg = plsc.load_gather(qrows, [e16, col])
                    return acc + kg * qg
                t = lax.fori_loop(0, DK, dot_step, jnp.zeros((16,), _f32))
                w = jnp.exp(t)
                plsc.store_scatter(wbuf, [e16, cbase16 + h], w)
        cv.wait()
        for g in range(BLK // DK):
            e16 = jax.lax.iota(jnp.int32, 16) + (g * 16)
            cbase16 = (dstv[pl.ds(g * 16, 16)] & 15) * HEADS
            for h in range(HEADS):
                w = plsc.load_gather(wbuf, [e16, cbase16 + h])
                for dk in range(DK):
                    col = jnp.full((16,), h * DK + dk, jnp.int32)
                    vg = plsc.load_gather(vbuf, [e16, col])
                    plsc.store_scatter(vbuf, [e16, col], w * vg)
        pltpu.sync_copy(vbuf, agg_sp.at[dstv], add=True)
        pltpu.sync_copy(wbuf, s_sp.at[dstrow], add=True)
        zero16 = jnp.zeros((16,), _f32)
        for g in range(BLK // DK):
            e16 = jax.lax.iota(jnp.int32, 16) + (g * 16)
            cbase16 = (dstv[pl.ds(g * 16, 16)] & 15) * HEADS
            for h in range(HEADS):
                plsc.store_scatter(wbuf, [e16, cbase16 + h], zero16)
        return carry

    lax.fori_loop(0, NBLK, blk, 0)
    plsc.subcore_barrier()

    cb = c * N
    cs = c * SROWS
    pltpu.sync_copy(agg_sp.at[pl.ds(r0, STRIPE)],
                    agg_out.at[pl.ds(cb + r0, STRIPE)])
    pltpu.sync_copy(agg_sp.at[pl.ds(TAIL0, TAIL)],
                    agg_out.at[pl.ds(cb + TAIL0, TAIL)])
    pltpu.sync_copy(s_sp.at[pl.ds(rs0, SSTRIPE)],
                    s_out.at[pl.ds(cs + rs0, SSTRIPE)])


@functools.lru_cache(maxsize=1)
def _get_edge_call():
  return pl.kernel(
    _edge_body,
    out_type=(
        jax.ShapeDtypeStruct((NC * N, D), _f32),
        jax.ShapeDtypeStruct((NC * SROWS, D), _f32),
    ),
    mesh=plsc.VectorSubcoreMesh(
        core_axis_name="c", subcore_axis_name="s",
        num_cores=NC, num_subcores=NS),
    scratch_types=[
        pltpu.MemorySpace.VMEM((BLK,), jnp.int32),
        pltpu.MemorySpace.VMEM((BLK,), jnp.int32),
        pltpu.MemorySpace.VMEM((BLK,), jnp.int32),
        pltpu.MemorySpace.VMEM((BLK, D), _f32),
        pltpu.MemorySpace.VMEM((BLK, D), _f32),
        pltpu.MemorySpace.VMEM((BLK, D), _f32),
        pltpu.MemorySpace.VMEM((BLK, D), _f32),
        pltpu.MemorySpace.VMEM_SHARED((N, D), _f32),
        pltpu.MemorySpace.VMEM_SHARED((SROWS, D), _f32),
        pltpu.SemaphoreType.DMA,
    ],
    compiler_params=pltpu.CompilerParams(needs_layout_passes=False),
  )


def _out_body(a_ref, s_ref, h_ref, w_ref, e_ref, ba_ref, beta_ref, o_ref):
    sfull = jnp.dot(s_ref[0] + s_ref[1], e_ref[...],
                    preferred_element_type=_f32)
    sfull = jnp.where(sfull > 0.0, sfull, 1.0)
    aggn = (a_ref[0] + a_ref[1]) / sfull
    o_ref[...] = (
        jnp.dot(aggn, w_ref[...], preferred_element_type=_f32)
        + ba_ref[...] + h_ref[...] * beta_ref[...]
    )


_out_call = pl.pallas_call(
    _out_body,
    grid=(N // RB,),
    in_specs=[
        pl.BlockSpec((NC, RB, D), lambda i: (0, i, 0)),
        pl.BlockSpec((NC, RB, HEADS), lambda i: (0, i, 0)),
        pl.BlockSpec((RB, D), lambda i: (i, 0)),
        pl.BlockSpec((D, D), lambda i: (0, 0)),
        pl.BlockSpec((HEADS, D), lambda i: (0, 0)),
        pl.BlockSpec((1, D), lambda i: (0, 0)),
        pl.BlockSpec((1, D), lambda i: (0, 0)),
    ],
    out_specs=pl.BlockSpec((RB, D), lambda i: (i, 0)),
    out_shape=jax.ShapeDtypeStruct((N, D), _f32),
)


def _fold(W, b, R):
    We = jnp.einsum('ihj,hjk->ihk', W.reshape(D, HEADS, DK), R).reshape(D, D)
    be = jnp.einsum('hj,hjk->hk', b.reshape(HEADS, DK), R).reshape(D)
    return We, be


def kernel(h_author, h_paper, edge_index_writes, edge_index_writtenby,
           Wk, bk, Wq, bq, Wv, bv, Wa, ba, rel_att, rel_msg, rel_pri, skip):
    h_author = h_author.astype(_f32)
    h_paper = h_paper.astype(_f32)

    Wk0, bk0 = _fold(Wk[0], bk[0], rel_att[0])
    Wv0, bv0 = _fold(Wv[0], bv[0], rel_msg[0])
    Wk1, bk1 = _fold(Wk[1], bk[1], rel_att[1])
    Wv1, bv1 = _fold(Wv[1], bv[1], rel_msg[1])
    scale0 = jnp.repeat(rel_pri[0] / SQRT_DK, DK)
    scale1 = jnp.repeat(rel_pri[1] / SQRT_DK, DK)
    Wq_a = Wq[0] * scale1[None, :]
    bq_a = bq[0] * scale1
    Wq_p = Wq[1] * scale0[None, :]
    bq_p = bq[1] * scale0

    Wstack = jnp.stack([
        jnp.concatenate([Wk0, Wv0, Wq_a], axis=1),
        jnp.concatenate([Wk1, Wv1, Wq_p], axis=1),
    ])
    bstack = jnp.stack([
        jnp.concatenate([bk0, bv0, bq_a]),
        jnp.concatenate([bk1, bv1, bq_p]),
    ])[:, None, :]
    hstack = jnp.stack([h_author, h_paper])

    kvq = _proj_call(hstack, Wstack, bstack)
    K_a, V_a, Q_a = kvq[0, :, :D], kvq[0, :, D:2 * D], kvq[0, :, 2 * D:]
    K_p, V_p, Q_p = kvq[1, :, :D], kvq[1, :, D:2 * D], kvq[1, :, 2 * D:]

    src0 = edge_index_writes[0].astype(jnp.int32)
    dst0 = edge_index_writes[1].astype(jnp.int32)
    src1 = edge_index_writtenby[0].astype(jnp.int32)
    dst1 = edge_index_writtenby[1].astype(jnp.int32)

    zagg = jnp.zeros((N, D), _f32)

    edge_call = _get_edge_call()
    agg_p2, sp_p2 = edge_call(K_a, Q_p, V_a, src0, dst0, zagg)
    agg_a2, sp_a2 = edge_call(K_p, Q_a, V_p, src1, dst1, zagg)
    agg_p2 = agg_p2.reshape(NC, N, D)
    agg_a2 = agg_a2.reshape(NC, N, D)
    s_p2 = sp_p2.reshape(NC, SROWS * 16, HEADS)[:, :N, :]
    s_a2 = sp_a2.reshape(NC, SROWS * 16, HEADS)[:, :N, :]

    emat = jnp.kron(jnp.eye(HEADS, dtype=_f32), jnp.ones((1, DK), _f32))
    alpha_a = jax.nn.sigmoid(skip[0])
    alpha_p = jax.nn.sigmoid(skip[1])
    new_author = _out_call(
        agg_a2, s_a2, h_author, Wa[0] * alpha_a,
        emat, (ba[0] * alpha_a)[None, :],
        jnp.full((1, D), 1.0 - alpha_a, _f32))
    new_paper = _out_call(
        agg_p2, s_p2, h_paper, Wa[1] * alpha_p,
        emat, (ba[1] * alpha_p)[None, :],
        jnp.full((1, D), 1.0 - alpha_p, _f32))
    return (new_author, new_paper)
